# Initial kernel scaffold; baseline (speedup 1.0000x reference)
import jax
import jax.numpy as jnp
from jax import lax
from jax.experimental import pallas as pl
from jax.experimental.pallas import tpu as pltpu
from jax.experimental.pallas import tpu_sc as plsc

N = 10000
E = 160000
D = 256
HALF = 128
NT = 16
EPT = E // NT
CH = 80
NCH = EPT // CH
RPT = N // NT
ZCH = 125
N2 = 10240
ZSP = N2 // NT

R = 400
NB = N // R


def _deg_body(ei_ref, deg_ref, sidx, didx, ones_v, hs, hd, zbuf):
    c = lax.axis_index("c")
    s = lax.axis_index("s")

    def fill(j, carry):
        zbuf[pl.ds(j * 16, 16)] = jnp.zeros((16,), jnp.float32)
        return carry
    lax.fori_loop(0, ZSP // 16, fill, 0)
    for j in range(CH // 16):
        ones_v[pl.ds(j * 16, 16)] = jnp.ones((16,), jnp.float32)
    pltpu.sync_copy(zbuf, hs.at[pl.ds(s * ZSP, ZSP)])
    pltpu.sync_copy(zbuf, hd.at[pl.ds(s * ZSP, ZSP)])
    plsc.subcore_barrier()

    def chunk(j, carry):
        base = s * EPT + j * CH
        pltpu.sync_copy(ei_ref.at[c, 0, pl.ds(base, CH)], sidx)
        pltpu.sync_copy(ei_ref.at[c, 1, pl.ds(base, CH)], didx)
        pltpu.sync_copy(ones_v, hs.at[sidx], add=True)
        pltpu.sync_copy(ones_v, hd.at[didx], add=True)
        return carry
    lax.fori_loop(0, NCH, chunk, 0)
    plsc.subcore_barrier()

    @pl.when(s == 0)
    def _():
        pltpu.sync_copy(hs.at[pl.ds(0, N)], deg_ref.at[2 * c])
        pltpu.sync_copy(hd.at[pl.ds(0, N)], deg_ref.at[2 * c + 1])


def _degrees(ei_all):
    return pl.kernel(
        _deg_body,
        out_type=jax.ShapeDtypeStruct((4, N), jnp.float32),
        mesh=plsc.VectorSubcoreMesh(core_axis_name="c", subcore_axis_name="s"),
        scratch_types=[
            pltpu.VMEM((CH,), jnp.int32),
            pltpu.VMEM((CH,), jnp.int32),
            pltpu.VMEM((CH,), jnp.float32),
            pltpu.VMEM_SHARED((N2,), jnp.float32),
            pltpu.VMEM_SHARED((N2,), jnp.float32),
            pltpu.VMEM((ZSP,), jnp.float32),
        ],
    )(ei_all)


def _scale_body(x_ref, dego_ref, degi_ref, xs_ref, nd_ref):
    do = dego_ref[:, :]
    ns = jnp.where(do > 0, lax.rsqrt(jnp.maximum(do, 1e-12)), 0.0)
    xs_ref[0] = x_ref[:, :] * ns
    di = degi_ref[:, :]
    nd_ref[:, :] = jnp.where(di > 0, lax.rsqrt(jnp.maximum(di, 1e-12)), 0.0)


def _scale(x, deg_out, deg_in):
    return pl.pallas_call(
        _scale_body,
        grid=(NB, 2),
        in_specs=[
            pl.BlockSpec((R, HALF), lambda i, h: (i, h)),
            pl.BlockSpec((R, 1), lambda i, h: (i, 0)),
            pl.BlockSpec((R, 1), lambda i, h: (i, 0)),
        ],
        out_specs=[
            pl.BlockSpec((1, R, HALF), lambda i, h: (h, i, 0)),
            pl.BlockSpec((R, 1), lambda i, h: (i, 0)),
        ],
        out_shape=[
            jax.ShapeDtypeStruct((2, N, HALF), jnp.float32),
            jax.ShapeDtypeStruct((N, 1), jnp.float32),
        ],
    )(x, deg_out, deg_in)


def _agg_body(xs_k_ref, src_k_ref, dst_k_ref, xs_l_ref, src_l_ref, dst_l_ref,
              out_k_ref, out_l_ref, sidx, didx, rows, agg, zbuf, sem):
    c = lax.axis_index("c")
    s = lax.axis_index("s")

    def fill(j, carry):
        zbuf[0, pl.ds(j * 16, 16)] = jnp.zeros((16,), jnp.float32)
        return carry
    lax.fori_loop(0, (ZCH * HALF) // 16, fill, 0)

    def run_rel(xs_ref, src_ref, dst_ref, out_ref):
        for q in range(RPT // ZCH):
            pltpu.sync_copy(zbuf, agg.at[pl.ds(s * RPT + q * ZCH, ZCH)])
        plsc.subcore_barrier()

        def chunk(j, carry):
            base = s * EPT + j * CH
            pltpu.sync_copy(src_ref.at[c, pl.ds(base, CH)], sidx)
            pltpu.sync_copy(dst_ref.at[pl.ds(base, CH)], didx)
            pltpu.async_copy(xs_ref.at[sidx], rows, sem).wait()
            pltpu.sync_copy(rows, agg.at[didx], add=True)
            return carry
        lax.fori_loop(0, NCH, chunk, 0)
        plsc.subcore_barrier()
        pltpu.sync_copy(agg.at[pl.ds(s * RPT, RPT)],
                        out_ref.at[c, pl.ds(s * RPT, RPT)])
        plsc.subcore_barrier()

    run_rel(xs_k_ref, src_k_ref, dst_k_ref, out_k_ref)
    run_rel(xs_l_ref, src_l_ref, dst_l_ref, out_l_ref)


def _aggregate(xs_k2, src2_k, dst_k, xs_l2, src2_l, dst_l):
    return pl.kernel(
        _agg_body,
        out_type=[
            jax.ShapeDtypeStruct((2, N, HALF), jnp.float32),
            jax.ShapeDtypeStruct((2, N, HALF), jnp.float32),
        ],
        mesh=plsc.VectorSubcoreMesh(core_axis_name="c", subcore_axis_name="s"),
        scratch_types=[
            pltpu.VMEM((CH,), jnp.int32),
            pltpu.VMEM((CH,), jnp.int32),
            pltpu.VMEM((CH, HALF), jnp.float32),
            pltpu.VMEM_SHARED((N, HALF), jnp.float32),
            pltpu.VMEM((ZCH, HALF), jnp.float32),
            pltpu.SemaphoreType.DMA,
        ],
    )(xs_k2, src2_k, dst_k, xs_l2, src2_l, dst_l)


def _out_body(ak_ref, al_ref, ndk_ref, ndl_ref, wk_ref, bk_ref, wl_ref,
              bl_ref, g_ref, be_ref, o_ref):
    ndk = ndk_ref[:, :]
    ndl = ndl_ref[:, :]
    hk = (jnp.dot(ak_ref[0] * ndk, wk_ref[0:HALF, :],
                  preferred_element_type=jnp.float32)
          + jnp.dot(ak_ref[1] * ndk, wk_ref[HALF:, :],
                    preferred_element_type=jnp.float32)
          + bk_ref[:, :])
    hl = (jnp.dot(al_ref[0] * ndl, wl_ref[0:HALF, :],
                  preferred_element_type=jnp.float32)
          + jnp.dot(al_ref[1] * ndl, wl_ref[HALF:, :],
                    preferred_element_type=jnp.float32)
          + bl_ref[:, :])
    hk = jnp.maximum(hk, 0.0)
    hl = jnp.maximum(hl, 0.0)
    m = jnp.maximum(hk, hl)
    lse = m + jnp.log(jnp.exp(hk - m) + jnp.exp(hl - m))
    mu = jnp.mean(lse, axis=1, keepdims=True)
    dv = lse - mu
    var = jnp.mean(dv * dv, axis=1, keepdims=True)
    o_ref[:, :] = dv * lax.rsqrt(var + 1e-6) * g_ref[:, :] + be_ref[:, :]


def _epilogue(agg_k, agg_l, nd_k, nd_l, wk, bk, wl, bl, gamma, beta):
    return pl.pallas_call(
        _out_body,
        grid=(NB,),
        in_specs=[
            pl.BlockSpec((2, R, HALF), lambda i: (0, i, 0)),
            pl.BlockSpec((2, R, HALF), lambda i: (0, i, 0)),
            pl.BlockSpec((R, 1), lambda i: (i, 0)),
            pl.BlockSpec((R, 1), lambda i: (i, 0)),
            pl.BlockSpec((D, D), lambda i: (0, 0)),
            pl.BlockSpec((1, D), lambda i: (0, 0)),
            pl.BlockSpec((D, D), lambda i: (0, 0)),
            pl.BlockSpec((1, D), lambda i: (0, 0)),
            pl.BlockSpec((1, D), lambda i: (0, 0)),
            pl.BlockSpec((1, D), lambda i: (0, 0)),
        ],
        out_specs=pl.BlockSpec((R, D), lambda i: (i, 0)),
        out_shape=jax.ShapeDtypeStruct((N, D), jnp.float32),
    )(agg_k, agg_l, nd_k, nd_l, wk, bk, wl, bl, gamma, beta)


def kernel(x_knows, x_likes, edge_index_knows, edge_index_likes, W_knows,
           b_knows, W_likes, b_likes, gamma, beta):
    ei_k = edge_index_knows.astype(jnp.int32)
    ei_l = edge_index_likes.astype(jnp.int32)
    ei_all = jnp.stack([ei_k, ei_l])

    deg = _degrees(ei_all)
    dego_k = deg[0].reshape(N, 1)
    degi_k = deg[1].reshape(N, 1)
    dego_l = deg[2].reshape(N, 1)
    degi_l = deg[3].reshape(N, 1)

    xs_k, nd_k = _scale(x_knows, dego_k, degi_k)
    xs_l, nd_l = _scale(x_likes, dego_l, degi_l)

    src_k = ei_k[0]
    src2_k = jnp.stack([src_k, src_k + N])
    src_l = ei_l[0]
    src2_l = jnp.stack([src_l, src_l + N])
    agg_k, agg_l = _aggregate(xs_k.reshape(2 * N, HALF), src2_k, ei_k[1],
                              xs_l.reshape(2 * N, HALF), src2_l, ei_l[1])

    bk = b_knows.reshape(1, D)
    bl = b_likes.reshape(1, D)
    return _epilogue(agg_k, agg_l, nd_k, nd_l, W_knows, bk, W_likes, bl,
                     gamma.reshape(1, D), beta.reshape(1, D))


# speedup vs baseline: 2.7716x; 2.7716x over previous
import jax
import jax.numpy as jnp
from jax import lax
from jax.experimental import pallas as pl
from jax.experimental.pallas import tpu as pltpu
from jax.experimental.pallas import tpu_sc as plsc

N = 10000
E = 160000
D = 256
HALF = 128
NT = 16
EPT = E // NT
CH = 80
NCH = EPT // CH
N2 = 10240
SP = N2 // NT

R = 400
NB = N // R


def _deg_body(idx_ref, deg_ref, sidx, didx, ones_v, hs, hd, zbuf):
    c = lax.axis_index("c")
    s = lax.axis_index("s")

    def fill(j, carry):
        zbuf[pl.ds(j * 16, 16)] = jnp.zeros((16,), jnp.float32)
        return carry
    lax.fori_loop(0, SP // 16, fill, 0)
    for j in range(CH // 16):
        ones_v[pl.ds(j * 16, 16)] = jnp.ones((16,), jnp.float32)
    pltpu.sync_copy(zbuf, hs.at[pl.ds(s * SP, SP)])
    pltpu.sync_copy(zbuf, hd.at[pl.ds(s * SP, SP)])
    plsc.subcore_barrier()

    rel = c * (2 * E)

    def chunk(j, carry):
        base = rel + s * EPT + j * CH
        pltpu.sync_copy(idx_ref.at[pl.ds(base, CH)], sidx)
        pltpu.sync_copy(idx_ref.at[pl.ds(base + E, CH)], didx)
        pltpu.sync_copy(ones_v, hs.at[sidx], add=True)
        pltpu.sync_copy(ones_v, hd.at[didx], add=True)
        return carry
    lax.fori_loop(0, NCH, chunk, 0)
    plsc.subcore_barrier()

    @pl.when(s == 0)
    def _():
        pltpu.sync_copy(hs, deg_ref.at[pl.ds(c * 2 * N2, N2)])
        pltpu.sync_copy(hd, deg_ref.at[pl.ds((c * 2 + 1) * N2, N2)])


def _degrees(idx_all):
    return pl.kernel(
        _deg_body,
        out_type=jax.ShapeDtypeStruct((4 * N2,), jnp.float32),
        mesh=plsc.VectorSubcoreMesh(core_axis_name="c", subcore_axis_name="s"),
        scratch_types=[
            pltpu.VMEM((CH,), jnp.int32),
            pltpu.VMEM((CH,), jnp.int32),
            pltpu.VMEM((CH,), jnp.float32),
            pltpu.VMEM_SHARED((N2,), jnp.float32),
            pltpu.VMEM_SHARED((N2,), jnp.float32),
            pltpu.VMEM((SP,), jnp.float32),
        ],
    )(idx_all)


def _scale_body(x_ref, dego_ref, degi_ref, xs_ref, nd_ref):
    do = dego_ref[:, :]
    ns = jnp.where(do > 0, lax.rsqrt(jnp.maximum(do, 1e-12)), 0.0)
    xs_ref[0] = x_ref[:, :] * ns
    di = degi_ref[:, :]
    nd_ref[:, :] = jnp.where(di > 0, lax.rsqrt(jnp.maximum(di, 1e-12)), 0.0)


def _scale(x, deg_out, deg_in):
    return pl.pallas_call(
        _scale_body,
        grid=(NB, 2),
        in_specs=[
            pl.BlockSpec((R, HALF), lambda i, h: (i, h)),
            pl.BlockSpec((R, 1), lambda i, h: (i, 0)),
            pl.BlockSpec((R, 1), lambda i, h: (i, 0)),
        ],
        out_specs=[
            pl.BlockSpec((1, R, HALF), lambda i, h: (h, i, 0)),
            pl.BlockSpec((R, 1), lambda i, h: (i, 0)),
        ],
        out_shape=[
            jax.ShapeDtypeStruct((2, N, HALF), jnp.float32),
            jax.ShapeDtypeStruct((N, 1), jnp.float32),
        ],
    )(x, deg_out, deg_in)


def _agg_body(xs_k_ref, src_k_ref, dst_k_ref, xs_l_ref, src_l_ref, dst_l_ref,
              out_k_ref, out_l_ref, sidx, didx, rows, agg, zbuf, sem):
    c = lax.axis_index("c")
    s = lax.axis_index("s")

    def fill(j, carry):
        zbuf[0, pl.ds(j * 16, 16)] = jnp.zeros((16,), jnp.float32)
        return carry
    lax.fori_loop(0, (128 * HALF) // 16, fill, 0)

    def run_rel(xs_ref, src_ref, dst_ref, out_ref):
        for q in range(SP // 128):
            pltpu.sync_copy(zbuf, agg.at[pl.ds(s * SP + q * 128, 128)])
        plsc.subcore_barrier()

        def chunk(j, carry):
            base = s * EPT + j * CH
            pltpu.sync_copy(src_ref.at[pl.ds(c * E + base, CH)], sidx)
            pltpu.sync_copy(dst_ref.at[pl.ds(base, CH)], didx)
            pltpu.async_copy(xs_ref.at[sidx], rows, sem).wait()
            pltpu.sync_copy(rows, agg.at[didx], add=True)
            return carry
        lax.fori_loop(0, NCH, chunk, 0)
        plsc.subcore_barrier()
        pltpu.sync_copy(agg.at[pl.ds(s * SP, SP)],
                        out_ref.at[c, pl.ds(s * SP, SP)])
        plsc.subcore_barrier()

    run_rel(xs_k_ref, src_k_ref, dst_k_ref, out_k_ref)
    run_rel(xs_l_ref, src_l_ref, dst_l_ref, out_l_ref)


def _aggregate(xs_k2, src2_k, dst_k, xs_l2, src2_l, dst_l):
    return pl.kernel(
        _agg_body,
        out_type=[
            jax.ShapeDtypeStruct((2, N2, HALF), jnp.float32),
            jax.ShapeDtypeStruct((2, N2, HALF), jnp.float32),
        ],
        mesh=plsc.VectorSubcoreMesh(core_axis_name="c", subcore_axis_name="s"),
        scratch_types=[
            pltpu.VMEM((CH,), jnp.int32),
            pltpu.VMEM((CH,), jnp.int32),
            pltpu.VMEM((CH, HALF), jnp.float32),
            pltpu.VMEM_SHARED((N2, HALF), jnp.float32),
            pltpu.VMEM((128, HALF), jnp.float32),
            pltpu.SemaphoreType.DMA,
        ],
    )(xs_k2, src2_k, dst_k, xs_l2, src2_l, dst_l)


def _out_body(ak_ref, al_ref, ndk_ref, ndl_ref, wk_ref, bk_ref, wl_ref,
              bl_ref, g_ref, be_ref, o_ref):
    ndk = ndk_ref[:, :]
    ndl = ndl_ref[:, :]
    hk = (jnp.dot(ak_ref[0] * ndk, wk_ref[0:HALF, :],
                  preferred_element_type=jnp.float32)
          + jnp.dot(ak_ref[1] * ndk, wk_ref[HALF:, :],
                    preferred_element_type=jnp.float32)
          + bk_ref[:, :])
    hl = (jnp.dot(al_ref[0] * ndl, wl_ref[0:HALF, :],
                  preferred_element_type=jnp.float32)
          + jnp.dot(al_ref[1] * ndl, wl_ref[HALF:, :],
                    preferred_element_type=jnp.float32)
          + bl_ref[:, :])
    hk = jnp.maximum(hk, 0.0)
    hl = jnp.maximum(hl, 0.0)
    m = jnp.maximum(hk, hl)
    lse = m + jnp.log(jnp.exp(hk - m) + jnp.exp(hl - m))
    mu = jnp.mean(lse, axis=1, keepdims=True)
    dv = lse - mu
    var = jnp.mean(dv * dv, axis=1, keepdims=True)
    o_ref[:, :] = dv * lax.rsqrt(var + 1e-6) * g_ref[:, :] + be_ref[:, :]


def _epilogue(agg_k, agg_l, nd_k, nd_l, wk, bk, wl, bl, gamma, beta):
    return pl.pallas_call(
        _out_body,
        grid=(NB,),
        in_specs=[
            pl.BlockSpec((2, R, HALF), lambda i: (0, i, 0)),
            pl.BlockSpec((2, R, HALF), lambda i: (0, i, 0)),
            pl.BlockSpec((R, 1), lambda i: (i, 0)),
            pl.BlockSpec((R, 1), lambda i: (i, 0)),
            pl.BlockSpec((D, D), lambda i: (0, 0)),
            pl.BlockSpec((1, D), lambda i: (0, 0)),
            pl.BlockSpec((D, D), lambda i: (0, 0)),
            pl.BlockSpec((1, D), lambda i: (0, 0)),
            pl.BlockSpec((1, D), lambda i: (0, 0)),
            pl.BlockSpec((1, D), lambda i: (0, 0)),
        ],
        out_specs=pl.BlockSpec((R, D), lambda i: (i, 0)),
        out_shape=jax.ShapeDtypeStruct((N, D), jnp.float32),
    )(agg_k, agg_l, nd_k, nd_l, wk, bk, wl, bl, gamma, beta)


def kernel(x_knows, x_likes, edge_index_knows, edge_index_likes, W_knows,
           b_knows, W_likes, b_likes, gamma, beta):
    ei_k = edge_index_knows.astype(jnp.int32)
    ei_l = edge_index_likes.astype(jnp.int32)
    idx_all = jnp.concatenate([ei_k[0], ei_k[1], ei_l[0], ei_l[1]])

    deg = _degrees(idx_all).reshape(4, N2)[:, :N]
    dego_k = deg[0].reshape(N, 1)
    degi_k = deg[1].reshape(N, 1)
    dego_l = deg[2].reshape(N, 1)
    degi_l = deg[3].reshape(N, 1)

    xs_k, nd_k = _scale(x_knows, dego_k, degi_k)
    xs_l, nd_l = _scale(x_likes, dego_l, degi_l)

    src2_k = jnp.concatenate([ei_k[0], ei_k[0] + N])
    src2_l = jnp.concatenate([ei_l[0], ei_l[0] + N])
    agg_k, agg_l = _aggregate(xs_k.reshape(2 * N, HALF), src2_k, ei_k[1],
                              xs_l.reshape(2 * N, HALF), src2_l, ei_l[1])

    bk = b_knows.reshape(1, D)
    bl = b_likes.reshape(1, D)
    return _epilogue(agg_k, agg_l, nd_k, nd_l, W_knows, bk, W_likes, bl,
                     gamma.reshape(1, D), beta.reshape(1, D))


# speedup vs baseline: 4.8574x; 1.7525x over previous
import jax
import jax.numpy as jnp
from jax import lax
from jax.experimental import pallas as pl
from jax.experimental.pallas import tpu as pltpu
from jax.experimental.pallas import tpu_sc as plsc

N = 10000
E = 160000
D = 256
HALF = 128
NT = 16
EPT = E // NT
CH = 80
NCH = EPT // CH
N2 = 10240
SP = N2 // NT

R = 400
NB = N // R


def _deg_body(idx_ref, deg_ref, sidx, didx, ones_v, hs, hd, zbuf):
    c = lax.axis_index("c")
    s = lax.axis_index("s")

    def fill(j, carry):
        zbuf[pl.ds(j * 16, 16)] = jnp.zeros((16,), jnp.float32)
        return carry
    lax.fori_loop(0, SP // 16, fill, 0)
    for j in range(CH // 16):
        ones_v[pl.ds(j * 16, 16)] = jnp.ones((16,), jnp.float32)
    pltpu.sync_copy(zbuf, hs.at[pl.ds(s * SP, SP)])
    pltpu.sync_copy(zbuf, hd.at[pl.ds(s * SP, SP)])
    plsc.subcore_barrier()

    rel = c * (2 * E)

    def chunk(j, carry):
        base = rel + s * EPT + j * CH
        pltpu.sync_copy(idx_ref.at[pl.ds(base, CH)], sidx)
        pltpu.sync_copy(idx_ref.at[pl.ds(base + E, CH)], didx)
        pltpu.sync_copy(ones_v, hs.at[sidx], add=True)
        pltpu.sync_copy(ones_v, hd.at[didx], add=True)
        return carry
    lax.fori_loop(0, NCH, chunk, 0)
    plsc.subcore_barrier()

    @pl.when(s == 0)
    def _():
        pltpu.sync_copy(hs, deg_ref.at[pl.ds(c * 2 * N2, N2)])
        pltpu.sync_copy(hd, deg_ref.at[pl.ds((c * 2 + 1) * N2, N2)])


def _degrees(idx_all):
    return pl.kernel(
        _deg_body,
        out_type=jax.ShapeDtypeStruct((4 * N2,), jnp.float32),
        mesh=plsc.VectorSubcoreMesh(core_axis_name="c", subcore_axis_name="s"),
        scratch_types=[
            pltpu.VMEM((CH,), jnp.int32),
            pltpu.VMEM((CH,), jnp.int32),
            pltpu.VMEM((CH,), jnp.float32),
            pltpu.VMEM_SHARED((N2,), jnp.float32),
            pltpu.VMEM_SHARED((N2,), jnp.float32),
            pltpu.VMEM((SP,), jnp.float32),
        ],
    )(idx_all)


def _scale_body(x_ref, dego_ref, degi_ref, xs_ref, nd_ref):
    do = dego_ref[:, :]
    ns = jnp.where(do > 0, lax.rsqrt(jnp.maximum(do, 1e-12)), 0.0)
    xs_ref[0] = x_ref[:, :] * ns
    di = degi_ref[:, :]
    nd_ref[:, :] = jnp.where(di > 0, lax.rsqrt(jnp.maximum(di, 1e-12)), 0.0)


def _scale(x, deg_out, deg_in):
    return pl.pallas_call(
        _scale_body,
        grid=(NB, 2),
        in_specs=[
            pl.BlockSpec((R, HALF), lambda i, h: (i, h)),
            pl.BlockSpec((R, 1), lambda i, h: (i, 0)),
            pl.BlockSpec((R, 1), lambda i, h: (i, 0)),
        ],
        out_specs=[
            pl.BlockSpec((1, R, HALF), lambda i, h: (h, i, 0)),
            pl.BlockSpec((R, 1), lambda i, h: (i, 0)),
        ],
        out_shape=[
            jax.ShapeDtypeStruct((2, N, HALF), jnp.float32),
            jax.ShapeDtypeStruct((N, 1), jnp.float32),
        ],
    )(x, deg_out, deg_in)


def _agg_body(xs_k_ref, src_k_ref, dst_k_ref, xs_l_ref, src_l_ref, dst_l_ref,
              out_k_ref, out_l_ref, sidx_v, didx0, didx1, rows0, rows1, agg,
              sem0, sem1, id0, id1):
    c = lax.axis_index("c")
    s = lax.axis_index("s")

    def run_rel(xs_ref, src_ref, dst_ref, out_ref):
        pltpu.sync_copy(src_ref.at[pl.ds(c * E + s * EPT, EPT)], sidx_v)
        def fill(j, carry):
            rows0[0, pl.ds(j * 16, 16)] = jnp.zeros((16,), jnp.float32)
            return carry
        lax.fori_loop(0, (CH * HALF) // 16, fill, 0)
        for q in range(SP // CH):
            pltpu.sync_copy(rows0, agg.at[pl.ds(s * SP + q * CH, CH)])
        plsc.subcore_barrier()

        def gather(j, buf, sem):
            return pltpu.async_copy(xs_ref.at[sidx_v.at[pl.ds(j * CH, CH)]],
                                    buf, sem)

        def dload(j, buf, sem):
            return pltpu.async_copy(dst_ref.at[pl.ds(s * EPT + j * CH, CH)],
                                    buf, sem)

        def dwait(j, buf, sem):
            pltpu.make_async_copy(dst_ref.at[pl.ds(s * EPT + j * CH, CH)],
                                  buf, sem).wait()

        def gwait(j, buf, sem):
            pltpu.make_async_copy(xs_ref.at[sidx_v.at[pl.ds(j * CH, CH)]],
                                  buf, sem).wait()

        def scatter(buf, dbuf):
            pltpu.sync_copy(buf, agg.at[dbuf], add=True)

        gather(0, rows0, sem0)
        dload(0, didx0, id0)

        def pair(i, carry):
            j = 2 * i
            gather(j + 1, rows1, sem1)
            dload(j + 1, didx1, id1)
            gwait(j, rows0, sem0)
            dwait(j, didx0, id0)
            scatter(rows0, didx0)
            gather(j + 2, rows0, sem0)
            dload(j + 2, didx0, id0)
            gwait(j + 1, rows1, sem1)
            dwait(j + 1, didx1, id1)
            scatter(rows1, didx1)
            return carry
        lax.fori_loop(0, (NCH - 1) // 2, pair, 0)
        gwait(NCH - 1, rows0, sem0)
        dwait(NCH - 1, didx0, id0)
        scatter(rows0, didx0)

        plsc.subcore_barrier()
        pltpu.sync_copy(agg.at[pl.ds(s * SP, SP)],
                        out_ref.at[c, pl.ds(s * SP, SP)])
        plsc.subcore_barrier()

    run_rel(xs_k_ref, src_k_ref, dst_k_ref, out_k_ref)
    run_rel(xs_l_ref, src_l_ref, dst_l_ref, out_l_ref)


def _aggregate(xs_k2, src2_k, dst_k, xs_l2, src2_l, dst_l):
    return pl.kernel(
        _agg_body,
        out_type=[
            jax.ShapeDtypeStruct((2, N2, HALF), jnp.float32),
            jax.ShapeDtypeStruct((2, N2, HALF), jnp.float32),
        ],
        mesh=plsc.VectorSubcoreMesh(core_axis_name="c", subcore_axis_name="s"),
        scratch_types=[
            pltpu.VMEM((EPT,), jnp.int32),
            pltpu.VMEM((CH,), jnp.int32),
            pltpu.VMEM((CH,), jnp.int32),
            pltpu.VMEM((CH, HALF), jnp.float32),
            pltpu.VMEM((CH, HALF), jnp.float32),
            pltpu.VMEM_SHARED((N2, HALF), jnp.float32),
            pltpu.SemaphoreType.DMA,
            pltpu.SemaphoreType.DMA,
            pltpu.SemaphoreType.DMA,
            pltpu.SemaphoreType.DMA,
        ],
    )(xs_k2, src2_k, dst_k, xs_l2, src2_l, dst_l)


def _out_body(ak_ref, al_ref, ndk_ref, ndl_ref, wk_ref, bk_ref, wl_ref,
              bl_ref, g_ref, be_ref, o_ref):
    ndk = ndk_ref[:, :]
    ndl = ndl_ref[:, :]
    hk = (jnp.dot(ak_ref[0] * ndk, wk_ref[0:HALF, :],
                  preferred_element_type=jnp.float32)
          + jnp.dot(ak_ref[1] * ndk, wk_ref[HALF:, :],
                    preferred_element_type=jnp.float32)
          + bk_ref[:, :])
    hl = (jnp.dot(al_ref[0] * ndl, wl_ref[0:HALF, :],
                  preferred_element_type=jnp.float32)
          + jnp.dot(al_ref[1] * ndl, wl_ref[HALF:, :],
                    preferred_element_type=jnp.float32)
          + bl_ref[:, :])
    hk = jnp.maximum(hk, 0.0)
    hl = jnp.maximum(hl, 0.0)
    m = jnp.maximum(hk, hl)
    lse = m + jnp.log(jnp.exp(hk - m) + jnp.exp(hl - m))
    mu = jnp.mean(lse, axis=1, keepdims=True)
    dv = lse - mu
    var = jnp.mean(dv * dv, axis=1, keepdims=True)
    o_ref[:, :] = dv * lax.rsqrt(var + 1e-6) * g_ref[:, :] + be_ref[:, :]


def _epilogue(agg_k, agg_l, nd_k, nd_l, wk, bk, wl, bl, gamma, beta):
    return pl.pallas_call(
        _out_body,
        grid=(NB,),
        in_specs=[
            pl.BlockSpec((2, R, HALF), lambda i: (0, i, 0)),
            pl.BlockSpec((2, R, HALF), lambda i: (0, i, 0)),
            pl.BlockSpec((R, 1), lambda i: (i, 0)),
            pl.BlockSpec((R, 1), lambda i: (i, 0)),
            pl.BlockSpec((D, D), lambda i: (0, 0)),
            pl.BlockSpec((1, D), lambda i: (0, 0)),
            pl.BlockSpec((D, D), lambda i: (0, 0)),
            pl.BlockSpec((1, D), lambda i: (0, 0)),
            pl.BlockSpec((1, D), lambda i: (0, 0)),
            pl.BlockSpec((1, D), lambda i: (0, 0)),
        ],
        out_specs=pl.BlockSpec((R, D), lambda i: (i, 0)),
        out_shape=jax.ShapeDtypeStruct((N, D), jnp.float32),
    )(agg_k, agg_l, nd_k, nd_l, wk, bk, wl, bl, gamma, beta)


def kernel(x_knows, x_likes, edge_index_knows, edge_index_likes, W_knows,
           b_knows, W_likes, b_likes, gamma, beta):
    ei_k = edge_index_knows.astype(jnp.int32)
    ei_l = edge_index_likes.astype(jnp.int32)
    idx_all = jnp.concatenate([ei_k[0], ei_k[1], ei_l[0], ei_l[1]])

    deg = _degrees(idx_all).reshape(4, N2)[:, :N]
    dego_k = deg[0].reshape(N, 1)
    degi_k = deg[1].reshape(N, 1)
    dego_l = deg[2].reshape(N, 1)
    degi_l = deg[3].reshape(N, 1)

    xs_k, nd_k = _scale(x_knows, dego_k, degi_k)
    xs_l, nd_l = _scale(x_likes, dego_l, degi_l)

    src2_k = jnp.concatenate([ei_k[0], ei_k[0] + N])
    src2_l = jnp.concatenate([ei_l[0], ei_l[0] + N])
    agg_k, agg_l = _aggregate(xs_k.reshape(2 * N, HALF), src2_k, ei_k[1],
                              xs_l.reshape(2 * N, HALF), src2_l, ei_l[1])

    bk = b_knows.reshape(1, D)
    bl = b_likes.reshape(1, D)
    return _epilogue(agg_k, agg_l, nd_k, nd_l, W_knows, bk, W_likes, bl,
                     gamma.reshape(1, D), beta.reshape(1, D))


# speedup vs baseline: 6.2993x; 1.2968x over previous
import jax
import jax.numpy as jnp
from jax import lax
from jax.experimental import pallas as pl
from jax.experimental.pallas import tpu as pltpu
from jax.experimental.pallas import tpu_sc as plsc

N = 10000
E = 160000
D = 256
HALF = 128
NT = 16
EPT = E // NT
CH = 80
NCH = EPT // CH
N2 = 10240
SP = N2 // NT

R = 400
NB = N // R


def _deg_body(idx_ref, deg_ref, sidx_v, didx_v, ones_v, hs, hd, zbuf, sem):
    c = lax.axis_index("c")
    s = lax.axis_index("s")

    def fill(j, carry):
        zbuf[pl.ds(j * 16, 16)] = jnp.zeros((16,), jnp.float32)
        return carry
    lax.fori_loop(0, SP // 16, fill, 0)
    for j in range(CH // 16):
        ones_v[pl.ds(j * 16, 16)] = jnp.ones((16,), jnp.float32)
    pltpu.sync_copy(zbuf, hs.at[pl.ds(s * SP, SP)])
    pltpu.sync_copy(zbuf, hd.at[pl.ds(s * SP, SP)])
    pltpu.sync_copy(idx_ref.at[2 * c, s], sidx_v)
    pltpu.sync_copy(idx_ref.at[2 * c + 1, s], didx_v)
    plsc.subcore_barrier()

    def issue(j, carry):
        pltpu.async_copy(ones_v, hs.at[sidx_v.at[j]], sem, add=True)
        pltpu.async_copy(ones_v, hd.at[didx_v.at[j]], sem, add=True)
        return carry
    lax.fori_loop(0, NCH, issue, 0)

    def drain(j, carry):
        pltpu.make_async_copy(ones_v, hs.at[sidx_v.at[0]], sem).wait()
        pltpu.make_async_copy(ones_v, hd.at[didx_v.at[0]], sem).wait()
        return carry
    lax.fori_loop(0, NCH, drain, 0)
    plsc.subcore_barrier()

    @pl.when(s == 0)
    def _():
        pltpu.sync_copy(hs, deg_ref.at[pl.ds(c * 2 * N2, N2)])
        pltpu.sync_copy(hd, deg_ref.at[pl.ds((c * 2 + 1) * N2, N2)])


def _degrees(idx_all):
    return pl.kernel(
        _deg_body,
        out_type=jax.ShapeDtypeStruct((4 * N2,), jnp.float32),
        mesh=plsc.VectorSubcoreMesh(core_axis_name="c", subcore_axis_name="s"),
        scratch_types=[
            pltpu.VMEM((NCH, CH), jnp.int32),
            pltpu.VMEM((NCH, CH), jnp.int32),
            pltpu.VMEM((CH,), jnp.float32),
            pltpu.VMEM_SHARED((N2,), jnp.float32),
            pltpu.VMEM_SHARED((N2,), jnp.float32),
            pltpu.VMEM((SP,), jnp.float32),
            pltpu.SemaphoreType.DMA,
        ],
    )(idx_all)


def _scale_body(x_ref, dego_ref, degi_ref, xs_ref, nd_ref):
    do = dego_ref[:, :]
    ns = jnp.where(do > 0, lax.rsqrt(jnp.maximum(do, 1e-12)), 0.0)
    xs_ref[0] = x_ref[:, :] * ns
    di = degi_ref[:, :]
    nd_ref[:, :] = jnp.where(di > 0, lax.rsqrt(jnp.maximum(di, 1e-12)), 0.0)


def _scale(x, deg_out, deg_in):
    return pl.pallas_call(
        _scale_body,
        grid=(NB, 2),
        in_specs=[
            pl.BlockSpec((R, HALF), lambda i, h: (i, h)),
            pl.BlockSpec((R, 1), lambda i, h: (i, 0)),
            pl.BlockSpec((R, 1), lambda i, h: (i, 0)),
        ],
        out_specs=[
            pl.BlockSpec((1, R, HALF), lambda i, h: (h, i, 0)),
            pl.BlockSpec((R, 1), lambda i, h: (i, 0)),
        ],
        out_shape=[
            jax.ShapeDtypeStruct((2, N, HALF), jnp.float32),
            jax.ShapeDtypeStruct((N, 1), jnp.float32),
        ],
    )(x, deg_out, deg_in)


def _agg_body(xs_k_ref, src_k_ref, dst_k_ref, xs_l_ref, src_l_ref, dst_l_ref,
              out_k_ref, out_l_ref, sidx_v, didx0, didx1, rows0, rows1, agg,
              sem0, sem1, id0, id1):
    c = lax.axis_index("c")
    s = lax.axis_index("s")

    def run_rel(xs_ref, src_ref, dst_ref, out_ref):
        pltpu.sync_copy(src_ref.at[pl.ds(c * E + s * EPT, EPT)], sidx_v)
        def fill(j, carry):
            rows0[0, pl.ds(j * 16, 16)] = jnp.zeros((16,), jnp.float32)
            return carry
        lax.fori_loop(0, (CH * HALF) // 16, fill, 0)
        for q in range(SP // CH):
            pltpu.sync_copy(rows0, agg.at[pl.ds(s * SP + q * CH, CH)])
        plsc.subcore_barrier()

        def gather(j, buf, sem):
            return pltpu.async_copy(xs_ref.at[sidx_v.at[pl.ds(j * CH, CH)]],
                                    buf, sem)

        def dload(j, buf, sem):
            return pltpu.async_copy(dst_ref.at[pl.ds(s * EPT + j * CH, CH)],
                                    buf, sem)

        def dwait(j, buf, sem):
            pltpu.make_async_copy(dst_ref.at[pl.ds(s * EPT + j * CH, CH)],
                                  buf, sem).wait()

        def gwait(j, buf, sem):
            pltpu.make_async_copy(xs_ref.at[sidx_v.at[pl.ds(j * CH, CH)]],
                                  buf, sem).wait()

        def scatter(buf, dbuf):
            pltpu.sync_copy(buf, agg.at[dbuf], add=True)

        gather(0, rows0, sem0)
        dload(0, didx0, id0)

        def pair(i, carry):
            j = 2 * i
            gather(j + 1, rows1, sem1)
            dload(j + 1, didx1, id1)
            gwait(j, rows0, sem0)
            dwait(j, didx0, id0)
            scatter(rows0, didx0)
            gather(j + 2, rows0, sem0)
            dload(j + 2, didx0, id0)
            gwait(j + 1, rows1, sem1)
            dwait(j + 1, didx1, id1)
            scatter(rows1, didx1)
            return carry
        lax.fori_loop(0, (NCH - 1) // 2, pair, 0)
        gwait(NCH - 1, rows0, sem0)
        dwait(NCH - 1, didx0, id0)
        scatter(rows0, didx0)

        plsc.subcore_barrier()
        pltpu.sync_copy(agg.at[pl.ds(s * SP, SP)],
                        out_ref.at[c, pl.ds(s * SP, SP)])
        plsc.subcore_barrier()

    run_rel(xs_k_ref, src_k_ref, dst_k_ref, out_k_ref)
    run_rel(xs_l_ref, src_l_ref, dst_l_ref, out_l_ref)


def _aggregate(xs_k2, src2_k, dst_k, xs_l2, src2_l, dst_l):
    return pl.kernel(
        _agg_body,
        out_type=[
            jax.ShapeDtypeStruct((2, N2, HALF), jnp.float32),
            jax.ShapeDtypeStruct((2, N2, HALF), jnp.float32),
        ],
        mesh=plsc.VectorSubcoreMesh(core_axis_name="c", subcore_axis_name="s"),
        scratch_types=[
            pltpu.VMEM((EPT,), jnp.int32),
            pltpu.VMEM((CH,), jnp.int32),
            pltpu.VMEM((CH,), jnp.int32),
            pltpu.VMEM((CH, HALF), jnp.float32),
            pltpu.VMEM((CH, HALF), jnp.float32),
            pltpu.VMEM_SHARED((N2, HALF), jnp.float32),
            pltpu.SemaphoreType.DMA,
            pltpu.SemaphoreType.DMA,
            pltpu.SemaphoreType.DMA,
            pltpu.SemaphoreType.DMA,
        ],
    )(xs_k2, src2_k, dst_k, xs_l2, src2_l, dst_l)


def _out_body(ak_ref, al_ref, ndk_ref, ndl_ref, wk_ref, bk_ref, wl_ref,
              bl_ref, g_ref, be_ref, o_ref):
    ndk = ndk_ref[:, :]
    ndl = ndl_ref[:, :]
    hk = (jnp.dot(ak_ref[0] * ndk, wk_ref[0:HALF, :],
                  preferred_element_type=jnp.float32)
          + jnp.dot(ak_ref[1] * ndk, wk_ref[HALF:, :],
                    preferred_element_type=jnp.float32)
          + bk_ref[:, :])
    hl = (jnp.dot(al_ref[0] * ndl, wl_ref[0:HALF, :],
                  preferred_element_type=jnp.float32)
          + jnp.dot(al_ref[1] * ndl, wl_ref[HALF:, :],
                    preferred_element_type=jnp.float32)
          + bl_ref[:, :])
    hk = jnp.maximum(hk, 0.0)
    hl = jnp.maximum(hl, 0.0)
    m = jnp.maximum(hk, hl)
    lse = m + jnp.log(jnp.exp(hk - m) + jnp.exp(hl - m))
    mu = jnp.mean(lse, axis=1, keepdims=True)
    dv = lse - mu
    var = jnp.mean(dv * dv, axis=1, keepdims=True)
    o_ref[:, :] = dv * lax.rsqrt(var + 1e-6) * g_ref[:, :] + be_ref[:, :]


def _epilogue(agg_k, agg_l, nd_k, nd_l, wk, bk, wl, bl, gamma, beta):
    return pl.pallas_call(
        _out_body,
        grid=(NB,),
        in_specs=[
            pl.BlockSpec((2, R, HALF), lambda i: (0, i, 0)),
            pl.BlockSpec((2, R, HALF), lambda i: (0, i, 0)),
            pl.BlockSpec((R, 1), lambda i: (i, 0)),
            pl.BlockSpec((R, 1), lambda i: (i, 0)),
            pl.BlockSpec((D, D), lambda i: (0, 0)),
            pl.BlockSpec((1, D), lambda i: (0, 0)),
            pl.BlockSpec((D, D), lambda i: (0, 0)),
            pl.BlockSpec((1, D), lambda i: (0, 0)),
            pl.BlockSpec((1, D), lambda i: (0, 0)),
            pl.BlockSpec((1, D), lambda i: (0, 0)),
        ],
        out_specs=pl.BlockSpec((R, D), lambda i: (i, 0)),
        out_shape=jax.ShapeDtypeStruct((N, D), jnp.float32),
    )(agg_k, agg_l, nd_k, nd_l, wk, bk, wl, bl, gamma, beta)


def kernel(x_knows, x_likes, edge_index_knows, edge_index_likes, W_knows,
           b_knows, W_likes, b_likes, gamma, beta):
    ei_k = edge_index_knows.astype(jnp.int32)
    ei_l = edge_index_likes.astype(jnp.int32)
    idx_all = jnp.stack([ei_k[0], ei_k[1], ei_l[0], ei_l[1]]
                        ).reshape(4, NT, NCH, CH)

    deg = _degrees(idx_all).reshape(4, N2)[:, :N]
    dego_k = deg[0].reshape(N, 1)
    degi_k = deg[1].reshape(N, 1)
    dego_l = deg[2].reshape(N, 1)
    degi_l = deg[3].reshape(N, 1)

    xs_k, nd_k = _scale(x_knows, dego_k, degi_k)
    xs_l, nd_l = _scale(x_likes, dego_l, degi_l)

    src2_k = jnp.concatenate([ei_k[0], ei_k[0] + N])
    src2_l = jnp.concatenate([ei_l[0], ei_l[0] + N])
    agg_k, agg_l = _aggregate(xs_k.reshape(2 * N, HALF), src2_k, ei_k[1],
                              xs_l.reshape(2 * N, HALF), src2_l, ei_l[1])

    bk = b_knows.reshape(1, D)
    bl = b_likes.reshape(1, D)
    return _epilogue(agg_k, agg_l, nd_k, nd_l, W_knows, bk, W_likes, bl,
                     gamma.reshape(1, D), beta.reshape(1, D))


# speedup vs baseline: 7.3863x; 1.1726x over previous
import jax
import jax.numpy as jnp
from jax import lax
from jax.experimental import pallas as pl
from jax.experimental.pallas import tpu as pltpu
from jax.experimental.pallas import tpu_sc as plsc

N = 10000
E = 160000
D = 256
HALF = 128
NT = 16
EPT = E // NT
CH = 80
NCH = EPT // CH
N2 = 10240
SP = N2 // NT

R = 400
NB = N // R


def _deg_body(idx_ref, deg_ref, sidx_v, didx_v, ones_v, hs, hd, zbuf, sem):
    c = lax.axis_index("c")
    s = lax.axis_index("s")

    def fill(j, carry):
        zbuf[pl.ds(j * 16, 16)] = jnp.zeros((16,), jnp.float32)
        return carry
    lax.fori_loop(0, SP // 16, fill, 0)
    for j in range(CH // 16):
        ones_v[pl.ds(j * 16, 16)] = jnp.ones((16,), jnp.float32)
    pltpu.sync_copy(zbuf, hs.at[pl.ds(s * SP, SP)])
    pltpu.sync_copy(zbuf, hd.at[pl.ds(s * SP, SP)])
    pltpu.sync_copy(idx_ref.at[2 * c, s], sidx_v)
    pltpu.sync_copy(idx_ref.at[2 * c + 1, s], didx_v)
    plsc.subcore_barrier()

    def issue(j, carry):
        pltpu.async_copy(ones_v, hs.at[sidx_v.at[j]], sem, add=True)
        pltpu.async_copy(ones_v, hd.at[didx_v.at[j]], sem, add=True)
        return carry
    lax.fori_loop(0, NCH, issue, 0)

    def drain(j, carry):
        pltpu.make_async_copy(ones_v, hs.at[sidx_v.at[0]], sem).wait()
        pltpu.make_async_copy(ones_v, hd.at[didx_v.at[0]], sem).wait()
        return carry
    lax.fori_loop(0, NCH, drain, 0)
    plsc.subcore_barrier()

    @pl.when(s == 0)
    def _():
        pltpu.sync_copy(hs, deg_ref.at[pl.ds(c * 2 * N2, N2)])
        pltpu.sync_copy(hd, deg_ref.at[pl.ds((c * 2 + 1) * N2, N2)])


def _degrees(idx_all):
    return pl.kernel(
        _deg_body,
        out_type=jax.ShapeDtypeStruct((4 * N2,), jnp.float32),
        mesh=plsc.VectorSubcoreMesh(core_axis_name="c", subcore_axis_name="s"),
        scratch_types=[
            pltpu.VMEM((NCH, CH), jnp.int32),
            pltpu.VMEM((NCH, CH), jnp.int32),
            pltpu.VMEM((CH,), jnp.float32),
            pltpu.VMEM_SHARED((N2,), jnp.float32),
            pltpu.VMEM_SHARED((N2,), jnp.float32),
            pltpu.VMEM((SP,), jnp.float32),
            pltpu.SemaphoreType.DMA,
        ],
    )(idx_all)


def _scale_body(xk_ref, xl_ref, dok_ref, dik_ref, dol_ref, dil_ref,
                xsk_ref, xsl_ref, ndk_ref, ndl_ref):
    def nrm(d):
        return jnp.where(d > 0, lax.rsqrt(jnp.maximum(d, 1e-12)), 0.0)

    xsk_ref[0] = xk_ref[:, :] * nrm(dok_ref[:, :])
    xsl_ref[0] = xl_ref[:, :] * nrm(dol_ref[:, :])
    ndk_ref[:, :] = nrm(dik_ref[:, :])
    ndl_ref[:, :] = nrm(dil_ref[:, :])


def _scale(x_knows, x_likes, dok, dik, dol, dil):
    return pl.pallas_call(
        _scale_body,
        grid=(NB, 2),
        in_specs=[
            pl.BlockSpec((R, HALF), lambda i, h: (i, h)),
            pl.BlockSpec((R, HALF), lambda i, h: (i, h)),
            pl.BlockSpec((R, 1), lambda i, h: (i, 0)),
            pl.BlockSpec((R, 1), lambda i, h: (i, 0)),
            pl.BlockSpec((R, 1), lambda i, h: (i, 0)),
            pl.BlockSpec((R, 1), lambda i, h: (i, 0)),
        ],
        out_specs=[
            pl.BlockSpec((1, R, HALF), lambda i, h: (h, i, 0)),
            pl.BlockSpec((1, R, HALF), lambda i, h: (h, i, 0)),
            pl.BlockSpec((R, 1), lambda i, h: (i, 0)),
            pl.BlockSpec((R, 1), lambda i, h: (i, 0)),
        ],
        out_shape=[
            jax.ShapeDtypeStruct((2, N, HALF), jnp.float32),
            jax.ShapeDtypeStruct((2, N, HALF), jnp.float32),
            jax.ShapeDtypeStruct((N, 1), jnp.float32),
            jax.ShapeDtypeStruct((N, 1), jnp.float32),
        ],
    )(x_knows, x_likes, dok, dik, dol, dil)


def _agg_body(xs_k_ref, src_k_ref, dst_k_ref, xs_l_ref, src_l_ref, dst_l_ref,
              out_k_ref, out_l_ref, s0, s1, s2, didx_v, b0, b1, b2, agg,
              i0, i1, i2, g0, g1, g2, t0, t1, t2):
    c = lax.axis_index("c")
    s = lax.axis_index("s")
    sbufs = (s0, s1, s2)
    isems = (i0, i1, i2)
    bufs = (b0, b1, b2)
    gs = (g0, g1, g2)
    ts = (t0, t1, t2)

    def run_rel(xs_ref, src_ref, dst_ref, out_ref):
        pltpu.sync_copy(dst_ref.at[s], didx_v)
        def fill(j, carry):
            b0[0, pl.ds(j * 16, 16)] = jnp.zeros((16,), jnp.float32)
            return carry
        lax.fori_loop(0, (CH * HALF) // 16, fill, 0)
        for q in range(SP // CH):
            pltpu.sync_copy(b0, agg.at[pl.ds(s * SP + q * CH, CH)])
        plsc.subcore_barrier()

        def iload(j, k):
            pltpu.async_copy(src_ref.at[pl.ds(c * E + s * EPT + j * CH, CH)],
                             sbufs[k], isems[k])

        def iwait(j, k):
            pltpu.make_async_copy(
                src_ref.at[pl.ds(c * E + s * EPT + j * CH, CH)],
                sbufs[k], isems[k]).wait()

        def gather(j, k):
            pltpu.async_copy(xs_ref.at[sbufs[k]], bufs[k], gs[k])

        def gwait(j, k):
            pltpu.make_async_copy(xs_ref.at[sbufs[k]], bufs[k], gs[k]).wait()

        def ascat(j, k):
            pltpu.async_copy(bufs[k], agg.at[didx_v.at[j]], ts[k], add=True)

        def twait(j, k):
            pltpu.make_async_copy(bufs[k], agg.at[didx_v.at[j]],
                                  ts[k]).wait()

        iload(0, 0)
        iload(1, 1)
        iload(2, 2)
        iwait(0, 0)
        gather(0, 0)
        iwait(1, 1)
        gather(1, 1)
        gwait(0, 0)
        ascat(0, 0)
        iload(3, 0)
        iwait(2, 2)
        gather(2, 2)
        gwait(1, 1)
        ascat(1, 1)

        def step(i, carry):
            for kk in range(3):
                j = 3 * i + 2 + kk
                k = (2 + kk) % 3
                kn = (k + 1) % 3
                iload(j + 2, (k + 2) % 3)
                twait(j - 2, kn)
                iwait(j + 1, kn)
                gather(j + 1, kn)
                gwait(j, k)
                ascat(j, k)
            return carry
        lax.fori_loop(0, (NCH - 4) // 3, step, 0)
        iload(NCH - 1, (NCH - 1) % 3)
        twait(NCH - 5, (NCH - 2) % 3)
        iwait(NCH - 2, (NCH - 2) % 3)
        gather(NCH - 2, (NCH - 2) % 3)
        gwait(NCH - 3, (NCH - 3) % 3)
        ascat(NCH - 3, (NCH - 3) % 3)
        twait(NCH - 4, (NCH - 1) % 3)
        iwait(NCH - 1, (NCH - 1) % 3)
        gather(NCH - 1, (NCH - 1) % 3)
        gwait(NCH - 2, (NCH - 2) % 3)
        ascat(NCH - 2, (NCH - 2) % 3)
        gwait(NCH - 1, (NCH - 1) % 3)
        ascat(NCH - 1, (NCH - 1) % 3)
        twait(NCH - 3, (NCH - 3) % 3)
        twait(NCH - 2, (NCH - 2) % 3)
        twait(NCH - 1, (NCH - 1) % 3)

        plsc.subcore_barrier()
        pltpu.sync_copy(agg.at[pl.ds(s * SP, SP)],
                        out_ref.at[c, pl.ds(s * SP, SP)])
        plsc.subcore_barrier()

    run_rel(xs_k_ref, src_k_ref, dst_k_ref, out_k_ref)
    run_rel(xs_l_ref, src_l_ref, dst_l_ref, out_l_ref)


def _aggregate(xs_k2, src2_k, dst_k, xs_l2, src2_l, dst_l):
    return pl.kernel(
        _agg_body,
        out_type=[
            jax.ShapeDtypeStruct((2, N2, HALF), jnp.float32),
            jax.ShapeDtypeStruct((2, N2, HALF), jnp.float32),
        ],
        mesh=plsc.VectorSubcoreMesh(core_axis_name="c", subcore_axis_name="s"),
        scratch_types=[
            pltpu.VMEM((CH,), jnp.int32),
            pltpu.VMEM((CH,), jnp.int32),
            pltpu.VMEM((CH,), jnp.int32),
            pltpu.VMEM((NCH, CH), jnp.int32),
            pltpu.VMEM((CH, HALF), jnp.float32),
            pltpu.VMEM((CH, HALF), jnp.float32),
            pltpu.VMEM((CH, HALF), jnp.float32),
            pltpu.VMEM_SHARED((N2, HALF), jnp.float32),
            pltpu.SemaphoreType.DMA,
            pltpu.SemaphoreType.DMA,
            pltpu.SemaphoreType.DMA,
            pltpu.SemaphoreType.DMA,
            pltpu.SemaphoreType.DMA,
            pltpu.SemaphoreType.DMA,
            pltpu.SemaphoreType.DMA,
            pltpu.SemaphoreType.DMA,
            pltpu.SemaphoreType.DMA,
        ],
    )(xs_k2, src2_k, dst_k, xs_l2, src2_l, dst_l)


def _out_body(ak_ref, al_ref, ndk_ref, ndl_ref, wk_ref, bk_ref, wl_ref,
              bl_ref, g_ref, be_ref, o_ref):
    ndk = ndk_ref[:, :]
    ndl = ndl_ref[:, :]
    hk = (jnp.dot(ak_ref[0] * ndk, wk_ref[0:HALF, :],
                  preferred_element_type=jnp.float32)
          + jnp.dot(ak_ref[1] * ndk, wk_ref[HALF:, :],
                    preferred_element_type=jnp.float32)
          + bk_ref[:, :])
    hl = (jnp.dot(al_ref[0] * ndl, wl_ref[0:HALF, :],
                  preferred_element_type=jnp.float32)
          + jnp.dot(al_ref[1] * ndl, wl_ref[HALF:, :],
                    preferred_element_type=jnp.float32)
          + bl_ref[:, :])
    hk = jnp.maximum(hk, 0.0)
    hl = jnp.maximum(hl, 0.0)
    m = jnp.maximum(hk, hl)
    lse = m + jnp.log(jnp.exp(hk - m) + jnp.exp(hl - m))
    mu = jnp.mean(lse, axis=1, keepdims=True)
    dv = lse - mu
    var = jnp.mean(dv * dv, axis=1, keepdims=True)
    o_ref[:, :] = dv * lax.rsqrt(var + 1e-6) * g_ref[:, :] + be_ref[:, :]


def _epilogue(agg_k, agg_l, nd_k, nd_l, wk, bk, wl, bl, gamma, beta):
    return pl.pallas_call(
        _out_body,
        grid=(NB,),
        in_specs=[
            pl.BlockSpec((2, R, HALF), lambda i: (0, i, 0)),
            pl.BlockSpec((2, R, HALF), lambda i: (0, i, 0)),
            pl.BlockSpec((R, 1), lambda i: (i, 0)),
            pl.BlockSpec((R, 1), lambda i: (i, 0)),
            pl.BlockSpec((D, D), lambda i: (0, 0)),
            pl.BlockSpec((1, D), lambda i: (0, 0)),
            pl.BlockSpec((D, D), lambda i: (0, 0)),
            pl.BlockSpec((1, D), lambda i: (0, 0)),
            pl.BlockSpec((1, D), lambda i: (0, 0)),
            pl.BlockSpec((1, D), lambda i: (0, 0)),
        ],
        out_specs=pl.BlockSpec((R, D), lambda i: (i, 0)),
        out_shape=jax.ShapeDtypeStruct((N, D), jnp.float32),
    )(agg_k, agg_l, nd_k, nd_l, wk, bk, wl, bl, gamma, beta)


def kernel(x_knows, x_likes, edge_index_knows, edge_index_likes, W_knows,
           b_knows, W_likes, b_likes, gamma, beta):
    ei_k = edge_index_knows.astype(jnp.int32)
    ei_l = edge_index_likes.astype(jnp.int32)
    idx_all = jnp.stack([ei_k[0], ei_k[1], ei_l[0], ei_l[1]]
                        ).reshape(4, NT, NCH, CH)

    deg = _degrees(idx_all).reshape(4, N2)[:, :N]
    dego_k = deg[0].reshape(N, 1)
    degi_k = deg[1].reshape(N, 1)
    dego_l = deg[2].reshape(N, 1)
    degi_l = deg[3].reshape(N, 1)

    xs_k, xs_l, nd_k, nd_l = _scale(x_knows, x_likes,
                                    dego_k, degi_k, dego_l, degi_l)

    src2_k = jnp.concatenate([ei_k[0], ei_k[0] + N])
    src2_l = jnp.concatenate([ei_l[0], ei_l[0] + N])
    agg_k, agg_l = _aggregate(xs_k.reshape(2 * N, HALF), src2_k,
                              ei_k[1].reshape(NT, NCH, CH),
                              xs_l.reshape(2 * N, HALF), src2_l,
                              ei_l[1].reshape(NT, NCH, CH))

    bk = b_knows.reshape(1, D)
    bl = b_likes.reshape(1, D)
    return _epilogue(agg_k, agg_l, nd_k, nd_l, W_knows, bk, W_likes, bl,
                     gamma.reshape(1, D), beta.reshape(1, D))


# speedup vs baseline: 7.9574x; 1.0773x over previous
import jax
import jax.numpy as jnp
from jax import lax
from jax.experimental import pallas as pl
from jax.experimental.pallas import tpu as pltpu
from jax.experimental.pallas import tpu_sc as plsc

N = 10000
E = 160000
D = 256
HALF = 128
NT = 16
EPT = E // NT
CH = 80
NCH = EPT // CH
N2 = 10240
SP = N2 // NT

R = 1000
NB = N // R


def _deg_body(idx_ref, deg_ref, sidx_v, didx_v, ones_v, hs, hd, zbuf, sem):
    c = lax.axis_index("c")
    s = lax.axis_index("s")

    def fill(j, carry):
        zbuf[pl.ds(j * 16, 16)] = jnp.zeros((16,), jnp.float32)
        return carry
    lax.fori_loop(0, SP // 16, fill, 0)
    for j in range(CH // 16):
        ones_v[pl.ds(j * 16, 16)] = jnp.ones((16,), jnp.float32)
    pltpu.sync_copy(zbuf, hs.at[pl.ds(s * SP, SP)])
    pltpu.sync_copy(zbuf, hd.at[pl.ds(s * SP, SP)])
    pltpu.sync_copy(idx_ref.at[2 * c, s], sidx_v)
    pltpu.sync_copy(idx_ref.at[2 * c + 1, s], didx_v)
    plsc.subcore_barrier()

    def issue(j, carry):
        pltpu.async_copy(ones_v, hs.at[sidx_v.at[j]], sem, add=True)
        pltpu.async_copy(ones_v, hd.at[didx_v.at[j]], sem, add=True)
        return carry
    lax.fori_loop(0, NCH, issue, 0)

    def drain(j, carry):
        pltpu.make_async_copy(ones_v, hs.at[sidx_v.at[0]], sem).wait()
        pltpu.make_async_copy(ones_v, hd.at[didx_v.at[0]], sem).wait()
        return carry
    lax.fori_loop(0, NCH, drain, 0)
    plsc.subcore_barrier()

    @pl.when(s == 0)
    def _():
        pltpu.sync_copy(hs, deg_ref.at[pl.ds(c * 2 * N2, N2)])
        pltpu.sync_copy(hd, deg_ref.at[pl.ds((c * 2 + 1) * N2, N2)])


def _degrees(idx_all):
    return pl.kernel(
        _deg_body,
        out_type=jax.ShapeDtypeStruct((4 * N2,), jnp.float32),
        mesh=plsc.VectorSubcoreMesh(core_axis_name="c", subcore_axis_name="s"),
        scratch_types=[
            pltpu.VMEM((NCH, CH), jnp.int32),
            pltpu.VMEM((NCH, CH), jnp.int32),
            pltpu.VMEM((CH,), jnp.float32),
            pltpu.VMEM_SHARED((N2,), jnp.float32),
            pltpu.VMEM_SHARED((N2,), jnp.float32),
            pltpu.VMEM((SP,), jnp.float32),
            pltpu.SemaphoreType.DMA,
        ],
    )(idx_all)


def _scale_body(xk_ref, xl_ref, dok_ref, dik_ref, dol_ref, dil_ref,
                xsk_ref, xsl_ref, ndk_ref, ndl_ref):
    def nrm(d):
        return jnp.where(d > 0, lax.rsqrt(jnp.maximum(d, 1e-12)), 0.0)

    xsk_ref[0] = xk_ref[:, :] * nrm(dok_ref[:, :])
    xsl_ref[0] = xl_ref[:, :] * nrm(dol_ref[:, :])
    ndk_ref[:, :] = nrm(dik_ref[:, :])
    ndl_ref[:, :] = nrm(dil_ref[:, :])


def _scale(x_knows, x_likes, dok, dik, dol, dil):
    return pl.pallas_call(
        _scale_body,
        grid=(NB, 2),
        in_specs=[
            pl.BlockSpec((R, HALF), lambda i, h: (i, h)),
            pl.BlockSpec((R, HALF), lambda i, h: (i, h)),
            pl.BlockSpec((R, 1), lambda i, h: (i, 0)),
            pl.BlockSpec((R, 1), lambda i, h: (i, 0)),
            pl.BlockSpec((R, 1), lambda i, h: (i, 0)),
            pl.BlockSpec((R, 1), lambda i, h: (i, 0)),
        ],
        out_specs=[
            pl.BlockSpec((1, R, HALF), lambda i, h: (h, i, 0)),
            pl.BlockSpec((1, R, HALF), lambda i, h: (h, i, 0)),
            pl.BlockSpec((R, 1), lambda i, h: (i, 0)),
            pl.BlockSpec((R, 1), lambda i, h: (i, 0)),
        ],
        out_shape=[
            jax.ShapeDtypeStruct((2, N, HALF), jnp.float32),
            jax.ShapeDtypeStruct((2, N, HALF), jnp.float32),
            jax.ShapeDtypeStruct((N, 1), jnp.float32),
            jax.ShapeDtypeStruct((N, 1), jnp.float32),
        ],
    )(x_knows, x_likes, dok, dik, dol, dil)


def _agg_body(xs_k_ref, src_k_ref, dst_k_ref, xs_l_ref, src_l_ref, dst_l_ref,
              out_k_ref, out_l_ref, s0, s1, s2, didx_v, b0, b1, b2, agg,
              i0, i1, i2, g0, g1, g2, t0, t1, t2):
    c = lax.axis_index("c")
    s = lax.axis_index("s")
    sbufs = (s0, s1, s2)
    isems = (i0, i1, i2)
    bufs = (b0, b1, b2)
    gs = (g0, g1, g2)
    ts = (t0, t1, t2)

    def run_rel(xs_ref, src_ref, dst_ref, out_ref):
        pltpu.sync_copy(dst_ref.at[s], didx_v)
        def fill(j, carry):
            b0[0, pl.ds(j * 16, 16)] = jnp.zeros((16,), jnp.float32)
            return carry
        lax.fori_loop(0, (CH * HALF) // 16, fill, 0)
        for q in range(SP // CH):
            pltpu.sync_copy(b0, agg.at[pl.ds(s * SP + q * CH, CH)])
        plsc.subcore_barrier()

        def iload(j, k):
            pltpu.async_copy(src_ref.at[pl.ds(c * E + s * EPT + j * CH, CH)],
                             sbufs[k], isems[k])

        def iwait(j, k):
            pltpu.make_async_copy(
                src_ref.at[pl.ds(c * E + s * EPT + j * CH, CH)],
                sbufs[k], isems[k]).wait()

        def gather(j, k):
            pltpu.async_copy(xs_ref.at[sbufs[k]], bufs[k], gs[k])

        def gwait(j, k):
            pltpu.make_async_copy(xs_ref.at[sbufs[k]], bufs[k], gs[k]).wait()

        def ascat(j, k):
            pltpu.async_copy(bufs[k], agg.at[didx_v.at[j]], ts[k], add=True)

        def twait(j, k):
            pltpu.make_async_copy(bufs[k], agg.at[didx_v.at[j]],
                                  ts[k]).wait()

        iload(0, 0)
        iload(1, 1)
        iload(2, 2)
        iwait(0, 0)
        gather(0, 0)
        iwait(1, 1)
        gather(1, 1)
        gwait(0, 0)
        ascat(0, 0)
        iload(3, 0)
        iwait(2, 2)
        gather(2, 2)
        gwait(1, 1)
        ascat(1, 1)

        def step(i, carry):
            for kk in range(3):
                j = 3 * i + 2 + kk
                k = (2 + kk) % 3
                kn = (k + 1) % 3
                iload(j + 2, (k + 2) % 3)
                twait(j - 2, kn)
                iwait(j + 1, kn)
                gather(j + 1, kn)
                gwait(j, k)
                ascat(j, k)
            return carry
        lax.fori_loop(0, (NCH - 4) // 3, step, 0)
        iload(NCH - 1, (NCH - 1) % 3)
        twait(NCH - 5, (NCH - 2) % 3)
        iwait(NCH - 2, (NCH - 2) % 3)
        gather(NCH - 2, (NCH - 2) % 3)
        gwait(NCH - 3, (NCH - 3) % 3)
        ascat(NCH - 3, (NCH - 3) % 3)
        twait(NCH - 4, (NCH - 1) % 3)
        iwait(NCH - 1, (NCH - 1) % 3)
        gather(NCH - 1, (NCH - 1) % 3)
        gwait(NCH - 2, (NCH - 2) % 3)
        ascat(NCH - 2, (NCH - 2) % 3)
        gwait(NCH - 1, (NCH - 1) % 3)
        ascat(NCH - 1, (NCH - 1) % 3)
        twait(NCH - 3, (NCH - 3) % 3)
        twait(NCH - 2, (NCH - 2) % 3)
        twait(NCH - 1, (NCH - 1) % 3)

        plsc.subcore_barrier()
        pltpu.sync_copy(agg.at[pl.ds(s * SP, SP)],
                        out_ref.at[c, pl.ds(s * SP, SP)])
        plsc.subcore_barrier()

    run_rel(xs_k_ref, src_k_ref, dst_k_ref, out_k_ref)
    run_rel(xs_l_ref, src_l_ref, dst_l_ref, out_l_ref)


def _aggregate(xs_k2, src2_k, dst_k, xs_l2, src2_l, dst_l):
    return pl.kernel(
        _agg_body,
        out_type=[
            jax.ShapeDtypeStruct((2, N2, HALF), jnp.float32),
            jax.ShapeDtypeStruct((2, N2, HALF), jnp.float32),
        ],
        mesh=plsc.VectorSubcoreMesh(core_axis_name="c", subcore_axis_name="s"),
        scratch_types=[
            pltpu.VMEM((CH,), jnp.int32),
            pltpu.VMEM((CH,), jnp.int32),
            pltpu.VMEM((CH,), jnp.int32),
            pltpu.VMEM((NCH, CH), jnp.int32),
            pltpu.VMEM((CH, HALF), jnp.float32),
            pltpu.VMEM((CH, HALF), jnp.float32),
            pltpu.VMEM((CH, HALF), jnp.float32),
            pltpu.VMEM_SHARED((N2, HALF), jnp.float32),
            pltpu.SemaphoreType.DMA,
            pltpu.SemaphoreType.DMA,
            pltpu.SemaphoreType.DMA,
            pltpu.SemaphoreType.DMA,
            pltpu.SemaphoreType.DMA,
            pltpu.SemaphoreType.DMA,
            pltpu.SemaphoreType.DMA,
            pltpu.SemaphoreType.DMA,
            pltpu.SemaphoreType.DMA,
        ],
    )(xs_k2, src2_k, dst_k, xs_l2, src2_l, dst_l)


def _out_body(ak_ref, al_ref, ndk_ref, ndl_ref, wk_ref, bk_ref, wl_ref,
              bl_ref, g_ref, be_ref, o_ref):
    ndk = ndk_ref[:, :]
    ndl = ndl_ref[:, :]
    hk = (jnp.dot(ak_ref[0] * ndk, wk_ref[0:HALF, :],
                  preferred_element_type=jnp.float32)
          + jnp.dot(ak_ref[1] * ndk, wk_ref[HALF:, :],
                    preferred_element_type=jnp.float32)
          + bk_ref[:, :])
    hl = (jnp.dot(al_ref[0] * ndl, wl_ref[0:HALF, :],
                  preferred_element_type=jnp.float32)
          + jnp.dot(al_ref[1] * ndl, wl_ref[HALF:, :],
                    preferred_element_type=jnp.float32)
          + bl_ref[:, :])
    hk = jnp.maximum(hk, 0.0)
    hl = jnp.maximum(hl, 0.0)
    m = jnp.maximum(hk, hl)
    lse = m + jnp.log(jnp.exp(hk - m) + jnp.exp(hl - m))
    mu = jnp.mean(lse, axis=1, keepdims=True)
    dv = lse - mu
    var = jnp.mean(dv * dv, axis=1, keepdims=True)
    o_ref[:, :] = dv * lax.rsqrt(var + 1e-6) * g_ref[:, :] + be_ref[:, :]


def _epilogue(agg_k, agg_l, nd_k, nd_l, wk, bk, wl, bl, gamma, beta):
    return pl.pallas_call(
        _out_body,
        grid=(NB,),
        in_specs=[
            pl.BlockSpec((2, R, HALF), lambda i: (0, i, 0)),
            pl.BlockSpec((2, R, HALF), lambda i: (0, i, 0)),
            pl.BlockSpec((R, 1), lambda i: (i, 0)),
            pl.BlockSpec((R, 1), lambda i: (i, 0)),
            pl.BlockSpec((D, D), lambda i: (0, 0)),
            pl.BlockSpec((1, D), lambda i: (0, 0)),
            pl.BlockSpec((D, D), lambda i: (0, 0)),
            pl.BlockSpec((1, D), lambda i: (0, 0)),
            pl.BlockSpec((1, D), lambda i: (0, 0)),
            pl.BlockSpec((1, D), lambda i: (0, 0)),
        ],
        out_specs=pl.BlockSpec((R, D), lambda i: (i, 0)),
        out_shape=jax.ShapeDtypeStruct((N, D), jnp.float32),
    )(agg_k, agg_l, nd_k, nd_l, wk, bk, wl, bl, gamma, beta)


def kernel(x_knows, x_likes, edge_index_knows, edge_index_likes, W_knows,
           b_knows, W_likes, b_likes, gamma, beta):
    ei_k = edge_index_knows.astype(jnp.int32)
    ei_l = edge_index_likes.astype(jnp.int32)
    idx_all = jnp.stack([ei_k[0], ei_k[1], ei_l[0], ei_l[1]]
                        ).reshape(4, NT, NCH, CH)

    deg = _degrees(idx_all).reshape(4, N2)[:, :N]
    dego_k = deg[0].reshape(N, 1)
    degi_k = deg[1].reshape(N, 1)
    dego_l = deg[2].reshape(N, 1)
    degi_l = deg[3].reshape(N, 1)

    xs_k, xs_l, nd_k, nd_l = _scale(x_knows, x_likes,
                                    dego_k, degi_k, dego_l, degi_l)

    src2_k = jnp.concatenate([ei_k[0], ei_k[0] + N])
    src2_l = jnp.concatenate([ei_l[0], ei_l[0] + N])
    agg_k, agg_l = _aggregate(xs_k.reshape(2 * N, HALF), src2_k,
                              ei_k[1].reshape(NT, NCH, CH),
                              xs_l.reshape(2 * N, HALF), src2_l,
                              ei_l[1].reshape(NT, NCH, CH))

    bk = b_knows.reshape(1, D)
    bl = b_likes.reshape(1, D)
    return _epilogue(agg_k, agg_l, nd_k, nd_l, W_knows, bk, W_likes, bl,
                     gamma.reshape(1, D), beta.reshape(1, D))
